# Initial kernel scaffold; baseline (speedup 1.0000x reference)
import jax
import jax.numpy as jnp
from jax.experimental import pallas as pl


def kernel(seq, sub_match):
    raise NotImplementedError("write your pallas kernel here")


# speedup vs baseline: 5.4848x; 5.4848x over previous
import functools

import jax
import jax.numpy as jnp
from jax import lax
from jax.experimental import pallas as pl
from jax.experimental.pallas import tpu as pltpu
from jax.experimental.pallas import tpu_sc as plsc

N = 320000
D = 128
NSEG = 10000
SEG_PAD = 10240

NC = 2
NS = 16
NW = NC * NS
ROWS_W = N // NW
CHUNK = 80
NCH = ROWS_W // CHUNK
SEG_T = SEG_PAD // NS

_mesh = plsc.VectorSubcoreMesh(
    core_axis_name="c", subcore_axis_name="s", num_cores=NC, num_subcores=NS
)


@functools.partial(
    pl.kernel,
    out_type=(
        jax.ShapeDtypeStruct((NC * SEG_PAD, D), jnp.float32),
        jax.ShapeDtypeStruct((NC * SEG_PAD,), jnp.float32),
    ),
    mesh=_mesh,
    scratch_types=(
        pltpu.VMEM((NCH, CHUNK), jnp.int32),
        pltpu.VMEM((CHUNK, D), jnp.float32),
        pltpu.VMEM((CHUNK,), jnp.float32),
        pltpu.VMEM_SHARED((SEG_PAD, D), jnp.float32),
        pltpu.VMEM_SHARED((SEG_PAD,), jnp.float32),
    ),
)
def _sc_segsum(seq_hbm, idx_hbm, zsum_hbm, zcnt_hbm, psum_hbm, pcnt_hbm,
               idx_v, row_v, ones_v, ssum, scnt):
    cid = lax.axis_index("c")
    sid = lax.axis_index("s")
    wid = cid * NS + sid

    one16 = jnp.full((16,), 1.0, dtype=jnp.float32)
    for i in range(CHUNK // 16):
        ones_v[pl.ds(i * 16, 16)] = one16

    zb = sid * SEG_T
    pltpu.sync_copy(zsum_hbm.at[pl.ds(zb, SEG_T)], ssum.at[pl.ds(zb, SEG_T)])
    pltpu.sync_copy(zcnt_hbm.at[pl.ds(zb, SEG_T)], scnt.at[pl.ds(zb, SEG_T)])

    pltpu.sync_copy(idx_hbm.at[wid], idx_v)
    plsc.subcore_barrier()

    row0 = wid * ROWS_W

    def step(j, carry):
        pltpu.sync_copy(seq_hbm.at[pl.ds(row0 + j * CHUNK, CHUNK)], row_v)
        ids = idx_v.at[j]
        pltpu.sync_copy(row_v, ssum.at[ids], add=True)
        pltpu.sync_copy(ones_v, scnt.at[ids], add=True)
        return carry

    lax.fori_loop(0, NCH, step, 0)
    plsc.subcore_barrier()

    ob = cid * SEG_PAD + sid * SEG_T
    pltpu.sync_copy(ssum.at[pl.ds(sid * SEG_T, SEG_T)],
                    psum_hbm.at[pl.ds(ob, SEG_T)])
    pltpu.sync_copy(scnt.at[pl.ds(sid * SEG_T, SEG_T)],
                    pcnt_hbm.at[pl.ds(ob, SEG_T)])


_BLK = 1024


def _combine_body(a_ref, b_ref, ca_ref, cb_ref, o_ref):
    den = ca_ref[...] + cb_ref[...] + 1.0
    o_ref[...] = (a_ref[...] + b_ref[...]) / den


def _combine(psum, pcnt2d):
    grid = (SEG_PAD // _BLK,)
    nb = SEG_PAD // _BLK
    return pl.pallas_call(
        _combine_body,
        grid=grid,
        in_specs=[
            pl.BlockSpec((_BLK, D), lambda i: (i, 0)),
            pl.BlockSpec((_BLK, D), lambda i: (nb + i, 0)),
            pl.BlockSpec((_BLK, 1), lambda i: (i, 0)),
            pl.BlockSpec((_BLK, 1), lambda i: (nb + i, 0)),
        ],
        out_specs=pl.BlockSpec((_BLK, D), lambda i: (i, 0)),
        out_shape=jax.ShapeDtypeStruct((NSEG, D), jnp.float32),
    )(psum, psum, pcnt2d, pcnt2d)


def kernel(seq, sub_match):
    idx2d = sub_match.reshape(NW, NCH, CHUNK)
    zsum = jnp.zeros((SEG_PAD, D), dtype=jnp.float32)
    zcnt = jnp.zeros((SEG_PAD,), dtype=jnp.float32)
    psum, pcnt = _sc_segsum(seq, idx2d, zsum, zcnt)
    return _combine(psum, pcnt.reshape(NC * SEG_PAD, 1))


# speedup vs baseline: 9.2400x; 1.6847x over previous
import functools

import jax
import jax.numpy as jnp
from jax import lax
from jax.experimental import pallas as pl
from jax.experimental.pallas import tpu as pltpu
from jax.experimental.pallas import tpu_sc as plsc

N = 320000
D = 128
NSEG = 10000
SEG_PAD = 10240

NC = 2
NS = 16
NW = NC * NS
ROWS_W = N // NW
CHUNK = 80
NCH = ROWS_W // CHUNK
NBUF = 3
CLAG = 4
NGRP = -(-NCH // NBUF)
SEG_T = SEG_PAD // NS

_mesh = plsc.VectorSubcoreMesh(
    core_axis_name="c", subcore_axis_name="s", num_cores=NC, num_subcores=NS
)


@functools.partial(
    pl.kernel,
    out_type=(
        jax.ShapeDtypeStruct((NC * SEG_PAD, D), jnp.float32),
        jax.ShapeDtypeStruct((NC * SEG_PAD,), jnp.float32),
    ),
    mesh=_mesh,
    scratch_types=(
        pltpu.VMEM((NCH, CHUNK), jnp.int32),
        tuple(pltpu.VMEM((CHUNK, D), jnp.float32) for _ in range(NBUF)),
        pltpu.VMEM((CHUNK,), jnp.float32),
        pltpu.VMEM((CHUNK,), jnp.float32),
        pltpu.VMEM_SHARED((SEG_PAD, D), jnp.float32),
        pltpu.VMEM_SHARED((SEG_PAD,), jnp.float32),
        tuple(pltpu.SemaphoreType.DMA for _ in range(NBUF)),
        tuple(pltpu.SemaphoreType.DMA for _ in range(NBUF)),
        pltpu.SemaphoreType.DMA,
    ),
)
def _sc_segsum(seq_hbm, idx_hbm, psum_hbm, pcnt_hbm,
               idx_v, row_bufs, ones_v, zeros_v, ssum, scnt,
               sems, ssems, csem):
    cid = lax.axis_index("c")
    sid = lax.axis_index("s")
    wid = cid * NS + sid

    one16 = jnp.full((16,), 1.0, dtype=jnp.float32)
    zero16 = jnp.zeros((16,), dtype=jnp.float32)
    for i in range(CHUNK // 16):
        ones_v[pl.ds(i * 16, 16)] = one16
        zeros_v[pl.ds(i * 16, 16)] = zero16
    for r in range(CHUNK):
        for c in range(D // 16):
            row_bufs[0][r, pl.ds(c * 16, 16)] = zero16

    for k in range(SEG_T // CHUNK):
        zb = sid * SEG_T + k * CHUNK
        pltpu.sync_copy(row_bufs[0], ssum.at[pl.ds(zb, CHUNK)])
        pltpu.sync_copy(zeros_v, scnt.at[pl.ds(zb, CHUNK)])

    pltpu.sync_copy(idx_hbm.at[wid], idx_v)

    row0 = wid * ROWS_W

    def load(j, b):
        pltpu.async_copy(
            seq_hbm.at[pl.ds(row0 + j * CHUNK, CHUNK)], row_bufs[b], sems[b]
        )

    def wait_load(b):
        pltpu.make_async_copy(
            seq_hbm.at[pl.ds(row0, CHUNK)], row_bufs[b], sems[b]
        ).wait()

    def wait_scatter(b):
        pltpu.make_async_copy(
            row_bufs[b], ssum.at[pl.ds(0, CHUNK)], ssems[b]
        ).wait()

    def wait_count():
        pltpu.make_async_copy(
            ones_v, scnt.at[pl.ds(0, CHUNK)], csem
        ).wait()

    for b in range(NBUF):
        load(b, b)
    plsc.subcore_barrier()

    def step(g, carry):
        for b in range(NBUF):
            j = g * NBUF + b

            @pl.when(j < NCH)
            def _():
                wait_load(b)
                ids = idx_v.at[j]
                pltpu.async_copy(row_bufs[b], ssum.at[ids], ssems[b], add=True)
                pltpu.async_copy(ones_v, scnt.at[ids], csem, add=True)

                @pl.when(j >= CLAG)
                def _():
                    wait_count()

                wait_scatter(b)

                @pl.when(j + NBUF < NCH)
                def _():
                    load(j + NBUF, b)

        return carry

    lax.fori_loop(0, NGRP, step, 0)

    for _ in range(CLAG):
        wait_count()
    plsc.subcore_barrier()

    ob = cid * SEG_PAD + sid * SEG_T
    pltpu.sync_copy(ssum.at[pl.ds(sid * SEG_T, SEG_T)],
                    psum_hbm.at[pl.ds(ob, SEG_T)])
    pltpu.sync_copy(scnt.at[pl.ds(sid * SEG_T, SEG_T)],
                    pcnt_hbm.at[pl.ds(ob, SEG_T)])


_BLK = 1024


def _combine_body(a_ref, b_ref, ca_ref, cb_ref, o_ref):
    den = ca_ref[...] + cb_ref[...] + 1.0
    o_ref[...] = (a_ref[...] + b_ref[...]) / den


def _combine(psum, pcnt2d):
    nb = SEG_PAD // _BLK
    return pl.pallas_call(
        _combine_body,
        grid=(nb,),
        in_specs=[
            pl.BlockSpec((_BLK, D), lambda i: (i, 0)),
            pl.BlockSpec((_BLK, D), lambda i: (nb + i, 0)),
            pl.BlockSpec((_BLK, 1), lambda i: (i, 0)),
            pl.BlockSpec((_BLK, 1), lambda i: (nb + i, 0)),
        ],
        out_specs=pl.BlockSpec((_BLK, D), lambda i: (i, 0)),
        out_shape=jax.ShapeDtypeStruct((NSEG, D), jnp.float32),
    )(psum, psum, pcnt2d, pcnt2d)


def kernel(seq, sub_match):
    idx2d = sub_match.reshape(NW, NCH, CHUNK)
    psum, pcnt = _sc_segsum(seq, idx2d)
    return _combine(psum, pcnt.reshape(NC * SEG_PAD, 1))


# speedup vs baseline: 9.3364x; 1.0104x over previous
import functools

import jax
import jax.numpy as jnp
from jax import lax
from jax.experimental import pallas as pl
from jax.experimental.pallas import tpu as pltpu
from jax.experimental.pallas import tpu_sc as plsc

N = 320000
D = 128
NSEG = 10000
SEG_PAD = 10240

NC = 2
NS = 16
NW = NC * NS
ROWS_W = N // NW
CHUNK = 80
NCH = ROWS_W // CHUNK
NCH_PAD = 128
NBUF = 4
CLAG = 4
GRP = 16
NGRP = NCH_PAD // GRP
SEG_T = SEG_PAD // NS

_mesh = plsc.VectorSubcoreMesh(
    core_axis_name="c", subcore_axis_name="s", num_cores=NC, num_subcores=NS
)


@functools.partial(
    pl.kernel,
    out_type=(
        jax.ShapeDtypeStruct((NC * SEG_PAD, D), jnp.float32),
        jax.ShapeDtypeStruct((NC * SEG_PAD,), jnp.float32),
    ),
    mesh=_mesh,
    scratch_types=(
        tuple(pltpu.VMEM((8, CHUNK), jnp.int32) for _ in range(2)),
        tuple(pltpu.VMEM((CHUNK, D), jnp.float32) for _ in range(NBUF)),
        pltpu.VMEM((CHUNK,), jnp.float32),
        pltpu.VMEM((CHUNK,), jnp.float32),
        pltpu.VMEM_SHARED((SEG_PAD, D), jnp.float32),
        pltpu.VMEM_SHARED((SEG_PAD,), jnp.float32),
        tuple(pltpu.SemaphoreType.DMA for _ in range(2)),
        tuple(pltpu.SemaphoreType.DMA for _ in range(NBUF)),
        tuple(pltpu.SemaphoreType.DMA for _ in range(NBUF)),
        pltpu.SemaphoreType.DMA,
    ),
)
def _sc_segsum(seq_hbm, idx_hbm, psum_hbm, pcnt_hbm,
               idx_bufs, row_bufs, ones_v, zeros_v, ssum, scnt,
               isems, sems, ssems, csem):
    cid = lax.axis_index("c")
    sid = lax.axis_index("s")
    wid = cid * NS + sid

    row0 = wid * ROWS_W

    def load_ids(blk8, half):
        pltpu.async_copy(
            idx_hbm.at[wid, pl.ds(blk8 * 8, 8)], idx_bufs[half], isems[half]
        )

    def wait_ids(half):
        pltpu.make_async_copy(
            idx_hbm.at[wid, pl.ds(0, 8)], idx_bufs[half], isems[half]
        ).wait()

    def load(j, b):
        pltpu.async_copy(
            seq_hbm.at[pl.ds(row0 + j * CHUNK, CHUNK)], row_bufs[b], sems[b]
        )

    def wait_load(b):
        pltpu.make_async_copy(
            seq_hbm.at[pl.ds(row0, CHUNK)], row_bufs[b], sems[b]
        ).wait()

    def wait_scatter(b):
        pltpu.make_async_copy(
            row_bufs[b], ssum.at[pl.ds(0, CHUNK)], ssems[b]
        ).wait()

    def wait_count():
        pltpu.make_async_copy(
            ones_v, scnt.at[pl.ds(0, CHUNK)], csem
        ).wait()

    load_ids(0, 0)
    for b in (1, 2, 3):
        load(b, b)

    one16 = jnp.full((16,), 1.0, dtype=jnp.float32)
    zero16 = jnp.zeros((16,), dtype=jnp.float32)
    for i in range(CHUNK // 16):
        ones_v[pl.ds(i * 16, 16)] = one16
        zeros_v[pl.ds(i * 16, 16)] = zero16
    for r in range(CHUNK):
        for c in range(D // 16):
            row_bufs[0][r, pl.ds(c * 16, 16)] = zero16

    for k in range(SEG_T // CHUNK):
        zb = sid * SEG_T + k * CHUNK
        pltpu.sync_copy(row_bufs[0], ssum.at[pl.ds(zb, CHUNK)])
        pltpu.sync_copy(zeros_v, scnt.at[pl.ds(zb, CHUNK)])

    load(0, 0)

    plsc.subcore_barrier()

    def step(g, carry):
        for k in range(GRP):
            j = g * GRP + k
            b = k % NBUF
            half = k // 8
            ids_row = k % 8

            if k == 0:
                wait_ids(0)
            if k == 8:
                wait_ids(1)
            if k == 4:
                pltpu.async_copy(
                    idx_hbm.at[wid, pl.ds(pl.multiple_of(g * GRP + 8, 8), 8)],
                    idx_bufs[1], isems[1],
                )
            if k == 12:
                @pl.when(g < NGRP - 1)
                def _():
                    pltpu.async_copy(
                        idx_hbm.at[wid,
                                   pl.ds(pl.multiple_of((g + 1) * GRP, 8), 8)],
                        idx_bufs[0], isems[0],
                    )

            @pl.when(j < NCH)
            def _():
                wait_load(b)
                ids = idx_bufs[half].at[ids_row]
                pltpu.async_copy(row_bufs[b], ssum.at[ids], ssems[b], add=True)
                pltpu.async_copy(ones_v, scnt.at[ids], csem, add=True)

                @pl.when(j >= CLAG)
                def _():
                    wait_count()

                wait_scatter(b)

                @pl.when(j + NBUF < NCH)
                def _():
                    load(j + NBUF, b)

        return carry

    lax.fori_loop(0, NGRP, step, 0)

    for _ in range(CLAG):
        wait_count()
    plsc.subcore_barrier()

    ob = cid * SEG_PAD + sid * SEG_T
    pltpu.sync_copy(ssum.at[pl.ds(sid * SEG_T, SEG_T)],
                    psum_hbm.at[pl.ds(ob, SEG_T)])
    pltpu.sync_copy(scnt.at[pl.ds(sid * SEG_T, SEG_T)],
                    pcnt_hbm.at[pl.ds(ob, SEG_T)])


_BLK = 1024


def _combine_body(a_ref, b_ref, ca_ref, cb_ref, o_ref):
    den = ca_ref[...] + cb_ref[...] + 1.0
    o_ref[...] = (a_ref[...] + b_ref[...]) / den


def _combine(psum, pcnt2d):
    nb = SEG_PAD // _BLK
    return pl.pallas_call(
        _combine_body,
        grid=(nb,),
        in_specs=[
            pl.BlockSpec((_BLK, D), lambda i: (i, 0)),
            pl.BlockSpec((_BLK, D), lambda i: (nb + i, 0)),
            pl.BlockSpec((_BLK, 1), lambda i: (i, 0)),
            pl.BlockSpec((_BLK, 1), lambda i: (nb + i, 0)),
        ],
        out_specs=pl.BlockSpec((_BLK, D), lambda i: (i, 0)),
        out_shape=jax.ShapeDtypeStruct((NSEG, D), jnp.float32),
    )(psum, psum, pcnt2d, pcnt2d)


def kernel(seq, sub_match):
    idx3 = sub_match.reshape(NW, NCH, CHUNK)
    idx3 = jnp.pad(idx3, ((0, 0), (0, NCH_PAD - NCH), (0, 0)))
    psum, pcnt = _sc_segsum(seq, idx3)
    return _combine(psum, pcnt.reshape(NC * SEG_PAD, 1))
